# baseline (speedup 1.0000x reference)
import jax
import jax.numpy as jnp
from jax.experimental import pallas as pl
from jax.experimental.pallas import tpu as pltpu

_B = 16384
_D_IN = 768
_D_LAT = 64
_K = 1024
_NQ = 3
_CW = 0.25
_T = 2048
_NS = 8
_NSTEP = _B // _T


def _sorted_unique_frac(v):
    rr = jax.lax.broadcasted_iota(jnp.int32, (128, 128), 0)
    cc = jax.lax.broadcasted_iota(jnp.int32, (128, 128), 1)
    j = 2
    while j <= 16384:
        s = j >> 1
        while s >= 1:
            if s < 128:
                shiftL = jnp.concatenate([v[:, s:], v[:, :s]], axis=1)
                shiftR = jnp.concatenate([v[:, 128 - s:], v[:, :128 - s]],
                                         axis=1)
                low = (cc & s) == 0
            else:
                S = s >> 7
                shiftL = jnp.concatenate([v[S:, :], v[:S, :]], axis=0)
                shiftR = jnp.concatenate([v[128 - S:, :], v[:128 - S, :]],
                                         axis=0)
                low = (rr & S) == 0
            partner = jnp.where(low, shiftL, shiftR)
            if j < 128:
                up = (cc & j) == 0
            elif j >= 16384:
                up = jnp.full((128, 128), True)
            else:
                up = (rr & (j >> 7)) == 0
            keep_min = jnp.logical_not(jnp.logical_xor(up, low))
            v = jnp.where(keep_min, jnp.minimum(v, partner),
                          jnp.maximum(v, partner))
            s >>= 1
        j <<= 1

    colprev = jnp.concatenate([v[:, :1], v[:, :127]], axis=1)
    rowlast = v[:, 127:128]
    rowlast_shift = jnp.concatenate([v[:1, :1], rowlast[:127, :]], axis=0)
    prev = jnp.where(cc == 0, rowlast_shift, colprev)
    neq = (v != prev).astype(jnp.float32)
    n = jnp.sum(jnp.sum(neq, axis=1, keepdims=True), axis=0, keepdims=True)
    return (n + 1.0) / jnp.float32(_B)


def _fused_body(x_ref,
                ew0, eb0, ew1, eb1, ew2, eb2, ew3, eb3,
                dw0, db0, dw1, db1, dw2, db2, dw3, db3,
                cba_ref, cbtm_ref, cs2_ref,
                en2_ref, recon_ref, qerr_ref, pu_ref,
                ids_scr):
    i = pl.program_id(0)
    _H = _T // _NS
    xs = [x_ref[pl.ds(s * _H, _H), :] for s in range(_NS)]

    hs = list(xs)
    for (w, b, act) in ((ew0, eb0, True), (ew1, eb1, True),
                        (ew2, eb2, True), (ew3, eb3, False)):
        hs = [jnp.dot(h, w[...], preferred_element_type=jnp.float32) + b[...]
              for h in hs]
        if act:
            hs = [jnp.maximum(h, 0.0) for h in hs]

    zs = hs
    ress = list(zs)
    norms2 = [[] for _ in range(_NS)]
    combineds = [None] * _NS
    qerr = None
    for l in range(_NQ):
        cba = cba_ref[l]
        cbtm = cbtm_ref[l]
        cs2 = cs2_ref[l]
        for s in range(_NS):
            res = ress[s]
            rs2 = jnp.sum(res * res, axis=1, keepdims=True)
            d = (rs2 + jnp.dot(res, cbtm,
                               preferred_element_type=jnp.float32)) + cs2
            ids = jnp.argmin(d, axis=1, keepdims=True)
            oh = (jax.lax.broadcasted_iota(jnp.int32, d.shape, 1) == ids)
            ea = jnp.dot(oh.astype(jnp.float32), cba,
                         preferred_element_type=jnp.float32)
            e = ea[:, :_D_LAT]
            diff = res - e
            qpart = jnp.sum(jnp.sum(diff * diff, axis=1, keepdims=True),
                            axis=0, keepdims=True)
            qerr = qpart if qerr is None else qerr + qpart
            norms2[s].append(ea[:, _D_LAT:_D_LAT + 1])
            combineds[s] = (ids if combineds[s] is None
                            else combineds[s] * _K + ids)
            ress[s] = diff

    gs = [z - res for (z, res) in zip(zs, ress)]
    for (w, b, act) in ((dw0, db0, True), (dw1, db1, True),
                        (dw2, db2, True), (dw3, db3, False)):
        wb = w[...].astype(jnp.bfloat16)
        gs = [jnp.dot(g.astype(jnp.bfloat16), wb,
                      preferred_element_type=jnp.float32) + b[...]
              for g in gs]
        if act:
            gs = [jnp.maximum(g, 0.0) for g in gs]

    rsum = None
    for s in range(_NS):
        g = gs[s]
        inv = jax.lax.rsqrt(jnp.maximum(
            jnp.sum(g * g, axis=1, keepdims=True), 1e-24))
        xh = g * inv
        rd = xh - xs[s]
        rp = jnp.sum(jnp.sum(rd * rd, axis=1, keepdims=True),
                     axis=0, keepdims=True)
        rsum = rp if rsum is None else rsum + rp

    for s in range(_NS):
        en2_ref[pl.ds(s * _H, _H), :] = jnp.sqrt(
            jnp.concatenate(norms2[s], axis=1))
    rows = _T // 128
    hrows = rows // _NS
    for s in range(_NS):
        ids_scr[pl.ds(i * rows + s * hrows, hrows), :] = (
            combineds[s].reshape(hrows, 128))

    @pl.when(i == 0)
    def _init():
        recon_ref[...] = rsum
        qerr_ref[...] = qerr

    @pl.when(i > 0)
    def _acc():
        recon_ref[...] = recon_ref[...] + rsum
        qerr_ref[...] = qerr_ref[...] + qerr

    @pl.when(i == _NSTEP - 1)
    def _finish():
        pu_ref[...] = _sorted_unique_frac(ids_scr[...])


def kernel(x, enc_W0, enc_b0, enc_W1, enc_b1, enc_W2, enc_b2, enc_W3, enc_b3,
           dec_W0, dec_b0, dec_W1, dec_b1, dec_W2, dec_b2, dec_W3, dec_b3,
           codebooks):
    ebs = [enc_b0.reshape(1, -1), enc_b1.reshape(1, -1),
           enc_b2.reshape(1, -1), enc_b3.reshape(1, -1)]
    dbs = [dec_b0.reshape(1, -1), dec_b1.reshape(1, -1),
           dec_b2.reshape(1, -1), dec_b3.reshape(1, -1)]
    cbtm = -2.0 * jnp.swapaxes(codebooks, 1, 2)
    cs2 = jnp.sum(codebooks * codebooks, axis=2)
    cba = jnp.concatenate([codebooks, cs2[:, :, None]], axis=2)
    cs2r = cs2[:, None, :]

    grid = (_NSTEP,)

    def _full(shape):
        nd = len(shape)
        return pl.BlockSpec(shape, lambda i, _nd=nd: (0,) * _nd)

    in_specs = [pl.BlockSpec((_T, _D_IN), lambda i: (i, 0))]
    ws = [enc_W0, enc_W1, enc_W2, enc_W3]
    dws = [dec_W0, dec_W1, dec_W2, dec_W3]
    operands = [x]
    for l in range(4):
        operands += [ws[l], ebs[l]]
        in_specs += [_full(ws[l].shape), _full(ebs[l].shape)]
    for l in range(4):
        operands += [dws[l], dbs[l]]
        in_specs += [_full(dws[l].shape), _full(dbs[l].shape)]
    operands += [cba, cbtm, cs2r]
    in_specs += [_full(cba.shape), _full(cbtm.shape), _full(cs2r.shape)]

    out_shape = [
        jax.ShapeDtypeStruct((_B, _NQ), jnp.float32),
        jax.ShapeDtypeStruct((1, 1), jnp.float32),
        jax.ShapeDtypeStruct((1, 1), jnp.float32),
        jax.ShapeDtypeStruct((1, 1), jnp.float32),
    ]
    out_specs = [
        pl.BlockSpec((_T, _NQ), lambda i: (i, 0)),
        pl.BlockSpec((1, 1), lambda i: (0, 0)),
        pl.BlockSpec((1, 1), lambda i: (0, 0)),
        pl.BlockSpec((1, 1), lambda i: (0, 0)),
    ]

    en2, recon_s, qerr_s, pu = pl.pallas_call(
        _fused_body,
        grid=grid,
        in_specs=in_specs,
        out_specs=out_specs,
        out_shape=out_shape,
        scratch_shapes=[pltpu.VMEM((128, 128), jnp.int32)],
    )(*operands)

    en = en2
    recon = recon_s[0, 0]
    q_loss = (1.0 + _CW) * (qerr_s[0, 0] / jnp.float32(_B * _D_LAT))
    loss = recon + q_loss
    return (loss, recon, q_loss, en, pu[0, 0])


# speedup vs baseline: 1.0095x; 1.0095x over previous
import jax
import jax.numpy as jnp
from jax.experimental import pallas as pl
from jax.experimental.pallas import tpu as pltpu

_B = 16384
_D_IN = 768
_D_LAT = 64
_K = 1024
_NQ = 3
_CW = 0.25
_T = 2048
_NS = 8
_NSTEP = _B // _T


def _sorted_unique_frac(v):
    rr = jax.lax.broadcasted_iota(jnp.int32, (128, 128), 0)
    cc = jax.lax.broadcasted_iota(jnp.int32, (128, 128), 1)
    j = 2
    while j <= 16384:
        s = j >> 1
        while s >= 1:
            if s < 128:
                shiftL = jnp.concatenate([v[:, s:], v[:, :s]], axis=1)
                shiftR = jnp.concatenate([v[:, 128 - s:], v[:, :128 - s]],
                                         axis=1)
                low = (cc & s) == 0
            else:
                S = s >> 7
                shiftL = jnp.concatenate([v[S:, :], v[:S, :]], axis=0)
                shiftR = jnp.concatenate([v[128 - S:, :], v[:128 - S, :]],
                                         axis=0)
                low = (rr & S) == 0
            partner = jnp.where(low, shiftL, shiftR)
            if j < 128:
                up = (cc & j) == 0
            elif j >= 16384:
                up = jnp.full((128, 128), True)
            else:
                up = (rr & (j >> 7)) == 0
            keep_min = jnp.logical_not(jnp.logical_xor(up, low))
            v = jnp.where(keep_min, jnp.minimum(v, partner),
                          jnp.maximum(v, partner))
            s >>= 1
        j <<= 1

    colprev = jnp.concatenate([v[:, :1], v[:, :127]], axis=1)
    rowlast = v[:, 127:128]
    rowlast_shift = jnp.concatenate([v[:1, :1], rowlast[:127, :]], axis=0)
    prev = jnp.where(cc == 0, rowlast_shift, colprev)
    neq = (v != prev).astype(jnp.float32)
    n = jnp.sum(jnp.sum(neq, axis=1, keepdims=True), axis=0, keepdims=True)
    return (n + 1.0) / jnp.float32(_B)


def _fused_body(x_ref,
                ew0, eb0, ew1, eb1, ew2, eb2, ew3, eb3,
                dw0, db0, dw1, db1, dw2, db2, dw3, db3,
                cb_ref,
                en2_ref, recon_ref, qerr_ref, pu_ref,
                ids_scr, cbtm_scr, cba_scr, cs2_scr):
    i = pl.program_id(0)

    @pl.when(i == 0)
    def _prep():
        for l in range(_NQ):
            cb = cb_ref[l]
            cbtm = -2.0 * jnp.swapaxes(cb, 0, 1)
            cbtm_scr[l] = cbtm
            cs2 = 0.25 * jnp.sum(cbtm * cbtm, axis=0, keepdims=True)
            cs2_scr[l] = cs2
            cba_scr[l, :, 0:_D_LAT] = cb
            cba_scr[l, :, _D_LAT:_D_LAT + 1] = jnp.swapaxes(cs2, 0, 1)
    _H = _T // _NS
    xs = [x_ref[pl.ds(s * _H, _H), :] for s in range(_NS)]

    hs = list(xs)
    for (w, b, act) in ((ew0, eb0, True), (ew1, eb1, True),
                        (ew2, eb2, True), (ew3, eb3, False)):
        hs = [jnp.dot(h, w[...], preferred_element_type=jnp.float32) + b[...]
              for h in hs]
        if act:
            hs = [jnp.maximum(h, 0.0) for h in hs]

    zs = hs
    ress = list(zs)
    norms2 = [[] for _ in range(_NS)]
    combineds = [None] * _NS
    qerr = None
    for l in range(_NQ):
        cba = cba_scr[l]
        cbtm = cbtm_scr[l]
        cs2 = cs2_scr[l]
        for s in range(_NS):
            res = ress[s]
            rs2 = jnp.sum(res * res, axis=1, keepdims=True)
            d = (rs2 + jnp.dot(res, cbtm,
                               preferred_element_type=jnp.float32)) + cs2
            ids = jnp.argmin(d, axis=1, keepdims=True)
            oh = (jax.lax.broadcasted_iota(jnp.int32, d.shape, 1) == ids)
            ea = jnp.dot(oh.astype(jnp.float32), cba,
                         preferred_element_type=jnp.float32)
            e = ea[:, :_D_LAT]
            diff = res - e
            qpart = jnp.sum(jnp.sum(diff * diff, axis=1, keepdims=True),
                            axis=0, keepdims=True)
            qerr = qpart if qerr is None else qerr + qpart
            norms2[s].append(ea[:, _D_LAT:_D_LAT + 1])
            combineds[s] = (ids if combineds[s] is None
                            else combineds[s] * _K + ids)
            ress[s] = diff

    gs = [z - res for (z, res) in zip(zs, ress)]
    for (w, b, act) in ((dw0, db0, True), (dw1, db1, True),
                        (dw2, db2, True), (dw3, db3, False)):
        wb = w[...].astype(jnp.bfloat16)
        gs = [jnp.dot(g.astype(jnp.bfloat16), wb,
                      preferred_element_type=jnp.float32) + b[...]
              for g in gs]
        if act:
            gs = [jnp.maximum(g, 0.0) for g in gs]

    rsum = None
    for s in range(_NS):
        g = gs[s]
        inv = jax.lax.rsqrt(jnp.maximum(
            jnp.sum(g * g, axis=1, keepdims=True), 1e-24))
        xh = g * inv
        rd = xh - xs[s]
        rp = jnp.sum(jnp.sum(rd * rd, axis=1, keepdims=True),
                     axis=0, keepdims=True)
        rsum = rp if rsum is None else rsum + rp

    for s in range(_NS):
        en2_ref[pl.ds(s * _H, _H), :] = jnp.sqrt(
            jnp.concatenate(norms2[s], axis=1))
    rows = _T // 128
    hrows = rows // _NS
    for s in range(_NS):
        ids_scr[pl.ds(i * rows + s * hrows, hrows), :] = (
            combineds[s].reshape(hrows, 128))

    @pl.when(i == 0)
    def _init():
        recon_ref[...] = rsum
        qerr_ref[...] = qerr

    @pl.when(i > 0)
    def _acc():
        recon_ref[...] = recon_ref[...] + rsum
        qerr_ref[...] = qerr_ref[...] + qerr

    @pl.when(i == _NSTEP - 1)
    def _finish():
        pu_ref[...] = _sorted_unique_frac(ids_scr[...])


def kernel(x, enc_W0, enc_b0, enc_W1, enc_b1, enc_W2, enc_b2, enc_W3, enc_b3,
           dec_W0, dec_b0, dec_W1, dec_b1, dec_W2, dec_b2, dec_W3, dec_b3,
           codebooks):
    ebs = [enc_b0.reshape(1, -1), enc_b1.reshape(1, -1),
           enc_b2.reshape(1, -1), enc_b3.reshape(1, -1)]
    dbs = [dec_b0.reshape(1, -1), dec_b1.reshape(1, -1),
           dec_b2.reshape(1, -1), dec_b3.reshape(1, -1)]
    grid = (_NSTEP,)

    def _full(shape):
        nd = len(shape)
        return pl.BlockSpec(shape, lambda i, _nd=nd: (0,) * _nd)

    in_specs = [pl.BlockSpec((_T, _D_IN), lambda i: (i, 0))]
    ws = [enc_W0, enc_W1, enc_W2, enc_W3]
    dws = [dec_W0, dec_W1, dec_W2, dec_W3]
    operands = [x]
    for l in range(4):
        operands += [ws[l], ebs[l]]
        in_specs += [_full(ws[l].shape), _full(ebs[l].shape)]
    for l in range(4):
        operands += [dws[l], dbs[l]]
        in_specs += [_full(dws[l].shape), _full(dbs[l].shape)]
    operands += [codebooks]
    in_specs += [_full(codebooks.shape)]

    out_shape = [
        jax.ShapeDtypeStruct((_B, _NQ), jnp.float32),
        jax.ShapeDtypeStruct((1, 1), jnp.float32),
        jax.ShapeDtypeStruct((1, 1), jnp.float32),
        jax.ShapeDtypeStruct((1, 1), jnp.float32),
    ]
    out_specs = [
        pl.BlockSpec((_T, _NQ), lambda i: (i, 0)),
        pl.BlockSpec((1, 1), lambda i: (0, 0)),
        pl.BlockSpec((1, 1), lambda i: (0, 0)),
        pl.BlockSpec((1, 1), lambda i: (0, 0)),
    ]

    en2, recon_s, qerr_s, pu = pl.pallas_call(
        _fused_body,
        grid=grid,
        in_specs=in_specs,
        out_specs=out_specs,
        out_shape=out_shape,
        scratch_shapes=[pltpu.VMEM((128, 128), jnp.int32),
                        pltpu.VMEM((_NQ, _D_LAT, _K), jnp.float32),
                        pltpu.VMEM((_NQ, _K, _D_LAT + 1), jnp.float32),
                        pltpu.VMEM((_NQ, 1, _K), jnp.float32)],
    )(*operands)

    en = en2
    recon = recon_s[0, 0]
    q_loss = (1.0 + _CW) * (qerr_s[0, 0] / jnp.float32(_B * _D_LAT))
    loss = recon + q_loss
    return (loss, recon, q_loss, en, pu[0, 0])


# speedup vs baseline: 1.0202x; 1.0107x over previous
import jax
import jax.numpy as jnp
from jax.experimental import pallas as pl
from jax.experimental.pallas import tpu as pltpu

_B = 16384
_D_IN = 768
_D_LAT = 64
_K = 1024
_NQ = 3
_CW = 0.25
_T = 4096
_NS = 16
_NSTEP = _B // _T


def _sorted_unique_frac(v):
    rr = jax.lax.broadcasted_iota(jnp.int32, (128, 128), 0)
    cc = jax.lax.broadcasted_iota(jnp.int32, (128, 128), 1)
    j = 2
    while j <= 16384:
        s = j >> 1
        while s >= 1:
            if s < 128:
                shiftL = jnp.concatenate([v[:, s:], v[:, :s]], axis=1)
                shiftR = jnp.concatenate([v[:, 128 - s:], v[:, :128 - s]],
                                         axis=1)
                low = (cc & s) == 0
            else:
                S = s >> 7
                shiftL = jnp.concatenate([v[S:, :], v[:S, :]], axis=0)
                shiftR = jnp.concatenate([v[128 - S:, :], v[:128 - S, :]],
                                         axis=0)
                low = (rr & S) == 0
            partner = jnp.where(low, shiftL, shiftR)
            if j < 128:
                up = (cc & j) == 0
            elif j >= 16384:
                up = jnp.full((128, 128), True)
            else:
                up = (rr & (j >> 7)) == 0
            keep_min = jnp.logical_not(jnp.logical_xor(up, low))
            v = jnp.where(keep_min, jnp.minimum(v, partner),
                          jnp.maximum(v, partner))
            s >>= 1
        j <<= 1

    colprev = jnp.concatenate([v[:, :1], v[:, :127]], axis=1)
    rowlast = v[:, 127:128]
    rowlast_shift = jnp.concatenate([v[:1, :1], rowlast[:127, :]], axis=0)
    prev = jnp.where(cc == 0, rowlast_shift, colprev)
    neq = (v != prev).astype(jnp.float32)
    n = jnp.sum(jnp.sum(neq, axis=1, keepdims=True), axis=0, keepdims=True)
    return (n + 1.0) / jnp.float32(_B)


def _fused_body(x_ref,
                ew0, eb0, ew1, eb1, ew2, eb2, ew3, eb3,
                dw0, db0, dw1, db1, dw2, db2, dw3, db3,
                cba_ref, cbtm_ref, cs2_ref,
                en2_ref, recon_ref, qerr_ref, pu_ref,
                ids_scr):
    i = pl.program_id(0)
    _H = _T // _NS
    xs = [x_ref[pl.ds(s * _H, _H), :] for s in range(_NS)]

    hs = list(xs)
    for (w, b, act) in ((ew0, eb0, True), (ew1, eb1, True),
                        (ew2, eb2, True), (ew3, eb3, False)):
        hs = [jnp.dot(h, w[...], preferred_element_type=jnp.float32) + b[...]
              for h in hs]
        if act:
            hs = [jnp.maximum(h, 0.0) for h in hs]

    zs = hs
    ress = list(zs)
    norms2 = [[] for _ in range(_NS)]
    combineds = [None] * _NS
    qerr = None
    for l in range(_NQ):
        cba = cba_ref[l]
        cbtm = cbtm_ref[l]
        cs2 = cs2_ref[l]
        for s in range(_NS):
            res = ress[s]
            rs2 = jnp.sum(res * res, axis=1, keepdims=True)
            d = (rs2 + jnp.dot(res, cbtm,
                               preferred_element_type=jnp.float32)) + cs2
            ids = jnp.argmin(d, axis=1, keepdims=True)
            oh = (jax.lax.broadcasted_iota(jnp.int32, d.shape, 1) == ids)
            ea = jnp.dot(oh.astype(jnp.float32), cba,
                         preferred_element_type=jnp.float32)
            e = ea[:, :_D_LAT]
            diff = res - e
            qpart = jnp.sum(jnp.sum(diff * diff, axis=1, keepdims=True),
                            axis=0, keepdims=True)
            qerr = qpart if qerr is None else qerr + qpart
            norms2[s].append(ea[:, _D_LAT:_D_LAT + 1])
            combineds[s] = (ids if combineds[s] is None
                            else combineds[s] * _K + ids)
            ress[s] = diff

    gs = [z - res for (z, res) in zip(zs, ress)]
    for (w, b, act) in ((dw0, db0, True), (dw1, db1, True),
                        (dw2, db2, True), (dw3, db3, False)):
        wb = w[...].astype(jnp.bfloat16)
        gs = [jnp.dot(g.astype(jnp.bfloat16), wb,
                      preferred_element_type=jnp.float32) + b[...]
              for g in gs]
        if act:
            gs = [jnp.maximum(g, 0.0) for g in gs]

    rsum = None
    for s in range(_NS):
        g = gs[s]
        inv = jax.lax.rsqrt(jnp.maximum(
            jnp.sum(g * g, axis=1, keepdims=True), 1e-24))
        xh = g * inv
        rd = xh - xs[s]
        rp = jnp.sum(jnp.sum(rd * rd, axis=1, keepdims=True),
                     axis=0, keepdims=True)
        rsum = rp if rsum is None else rsum + rp

    for s in range(_NS):
        en2_ref[pl.ds(s * _H, _H), :] = jnp.sqrt(
            jnp.concatenate(norms2[s], axis=1))
    rows = _T // 128
    hrows = rows // _NS
    for s in range(_NS):
        ids_scr[pl.ds(i * rows + s * hrows, hrows), :] = (
            combineds[s].reshape(hrows, 128))

    @pl.when(i == 0)
    def _init():
        recon_ref[...] = rsum
        qerr_ref[...] = qerr

    @pl.when(i > 0)
    def _acc():
        recon_ref[...] = recon_ref[...] + rsum
        qerr_ref[...] = qerr_ref[...] + qerr

    @pl.when(i == _NSTEP - 1)
    def _finish():
        pu_ref[...] = _sorted_unique_frac(ids_scr[...])


def kernel(x, enc_W0, enc_b0, enc_W1, enc_b1, enc_W2, enc_b2, enc_W3, enc_b3,
           dec_W0, dec_b0, dec_W1, dec_b1, dec_W2, dec_b2, dec_W3, dec_b3,
           codebooks):
    ebs = [enc_b0.reshape(1, -1), enc_b1.reshape(1, -1),
           enc_b2.reshape(1, -1), enc_b3.reshape(1, -1)]
    dbs = [dec_b0.reshape(1, -1), dec_b1.reshape(1, -1),
           dec_b2.reshape(1, -1), dec_b3.reshape(1, -1)]
    cbtm = -2.0 * jnp.swapaxes(codebooks, 1, 2)
    cs2 = jnp.sum(codebooks * codebooks, axis=2)
    cba = jnp.concatenate([codebooks, cs2[:, :, None]], axis=2)
    cs2r = cs2[:, None, :]

    grid = (_NSTEP,)

    def _full(shape):
        nd = len(shape)
        return pl.BlockSpec(shape, lambda i, _nd=nd: (0,) * _nd)

    in_specs = [pl.BlockSpec((_T, _D_IN), lambda i: (i, 0))]
    ws = [enc_W0, enc_W1, enc_W2, enc_W3]
    dws = [dec_W0, dec_W1, dec_W2, dec_W3]
    operands = [x]
    for l in range(4):
        operands += [ws[l], ebs[l]]
        in_specs += [_full(ws[l].shape), _full(ebs[l].shape)]
    for l in range(4):
        operands += [dws[l], dbs[l]]
        in_specs += [_full(dws[l].shape), _full(dbs[l].shape)]
    operands += [cba, cbtm, cs2r]
    in_specs += [_full(cba.shape), _full(cbtm.shape), _full(cs2r.shape)]

    out_shape = [
        jax.ShapeDtypeStruct((_B, _NQ), jnp.float32),
        jax.ShapeDtypeStruct((1, 1), jnp.float32),
        jax.ShapeDtypeStruct((1, 1), jnp.float32),
        jax.ShapeDtypeStruct((1, 1), jnp.float32),
    ]
    out_specs = [
        pl.BlockSpec((_T, _NQ), lambda i: (i, 0)),
        pl.BlockSpec((1, 1), lambda i: (0, 0)),
        pl.BlockSpec((1, 1), lambda i: (0, 0)),
        pl.BlockSpec((1, 1), lambda i: (0, 0)),
    ]

    en2, recon_s, qerr_s, pu = pl.pallas_call(
        _fused_body,
        grid=grid,
        in_specs=in_specs,
        out_specs=out_specs,
        out_shape=out_shape,
        scratch_shapes=[pltpu.VMEM((128, 128), jnp.int32)],
    )(*operands)

    en = en2
    recon = recon_s[0, 0]
    q_loss = (1.0 + _CW) * (qerr_s[0, 0] / jnp.float32(_B * _D_LAT))
    loss = recon + q_loss
    return (loss, recon, q_loss, en, pu[0, 0])
